# baseline (speedup 1.0000x reference)
import jax
import jax.numpy as jnp
from jax.experimental import pallas as pl


def _add_block(x_ref, sp_ref, tp_ref, o_ref):
    o_ref[...] = x_ref[...] + sp_ref[...] + tp_ref[...]


def kernel(x, spatial_table, temporal_table):
    batch, seqlen, height, width, d = x.shape
    hw = height * width
    x4 = x.reshape(batch, seqlen, hw, d)
    tt3 = temporal_table.reshape(temporal_table.shape[0], 1, d)

    out = pl.pallas_call(
        _add_block,
        grid=(batch, seqlen),
        in_specs=[
            pl.BlockSpec((1, 1, hw, d), lambda b, t: (b, t, 0, 0)),
            pl.BlockSpec((hw, d), lambda b, t: (0, 0)),
            pl.BlockSpec((1, 1, d), lambda b, t: (t, 0, 0)),
        ],
        out_specs=pl.BlockSpec((1, 1, hw, d), lambda b, t: (b, t, 0, 0)),
        out_shape=jax.ShapeDtypeStruct((batch, seqlen, hw, d), x.dtype),
    )(x4, spatial_table, tt3)

    return out


# speedup vs baseline: 1.1605x; 1.1605x over previous
import jax
import jax.numpy as jnp
from jax.experimental import pallas as pl


def _add_block(x_ref, sp_ref, tp_ref, o_ref):
    o_ref[...] = x_ref[...] + sp_ref[...] + tp_ref[...]


def kernel(x, spatial_table, temporal_table):
    batch, seqlen, height, width, d = x.shape
    hw = height * width
    x4 = x.reshape(batch, seqlen, hw, d)
    tt3 = temporal_table.reshape(temporal_table.shape[0], 1, d)

    tchunk = 4
    out = pl.pallas_call(
        _add_block,
        grid=(batch, seqlen // tchunk),
        in_specs=[
            pl.BlockSpec((1, tchunk, hw, d), lambda b, t: (b, t, 0, 0)),
            pl.BlockSpec((hw, d), lambda b, t: (0, 0)),
            pl.BlockSpec((tchunk, 1, d), lambda b, t: (t, 0, 0)),
        ],
        out_specs=pl.BlockSpec((1, tchunk, hw, d), lambda b, t: (b, t, 0, 0)),
        out_shape=jax.ShapeDtypeStruct((batch, seqlen, hw, d), x.dtype),
    )(x4, spatial_table, tt3)

    return out


# speedup vs baseline: 1.1751x; 1.0126x over previous
import jax
import jax.numpy as jnp
from jax.experimental import pallas as pl


def _add_block(x_ref, sp_ref, tp_ref, o_ref):
    o_ref[...] = x_ref[...] + sp_ref[...] + tp_ref[...]


def kernel(x, spatial_table, temporal_table):
    batch, seqlen, height, width, d = x.shape
    hw = height * width
    x4 = x.reshape(batch, seqlen, hw, d)
    tt3 = temporal_table.reshape(temporal_table.shape[0], 1, d)

    tchunk = 8
    out = pl.pallas_call(
        _add_block,
        grid=(batch, seqlen // tchunk),
        in_specs=[
            pl.BlockSpec((1, tchunk, hw, d), lambda b, t: (b, t, 0, 0)),
            pl.BlockSpec((hw, d), lambda b, t: (0, 0)),
            pl.BlockSpec((tchunk, 1, d), lambda b, t: (t, 0, 0)),
        ],
        out_specs=pl.BlockSpec((1, tchunk, hw, d), lambda b, t: (b, t, 0, 0)),
        out_shape=jax.ShapeDtypeStruct((batch, seqlen, hw, d), x.dtype),
    )(x4, spatial_table, tt3)

    return out


# speedup vs baseline: 1.1786x; 1.0029x over previous
import jax
import jax.numpy as jnp
from jax.experimental import pallas as pl
from jax.experimental.pallas import tpu as pltpu


def _add_block(x_ref, sp_ref, tp_ref, o_ref):
    o_ref[...] = x_ref[...] + sp_ref[...] + tp_ref[...]


def kernel(x, spatial_table, temporal_table):
    batch, seqlen, height, width, d = x.shape
    hw = height * width
    x4 = x.reshape(batch, seqlen, hw, d)
    tt3 = temporal_table.reshape(temporal_table.shape[0], 1, d)

    tchunk = 8
    out = pl.pallas_call(
        _add_block,
        compiler_params=pltpu.CompilerParams(
            vmem_limit_bytes=64 * 1024 * 1024,
        ),
        grid=(batch, seqlen // tchunk),
        in_specs=[
            pl.BlockSpec((1, tchunk, hw, d), lambda b, t: (b, t, 0, 0)),
            pl.BlockSpec((hw, d), lambda b, t: (0, 0)),
            pl.BlockSpec((tchunk, 1, d), lambda b, t: (t, 0, 0)),
        ],
        out_specs=pl.BlockSpec((1, tchunk, hw, d), lambda b, t: (b, t, 0, 0)),
        out_shape=jax.ShapeDtypeStruct((batch, seqlen, hw, d), x.dtype),
    )(x4, spatial_table, tt3)

    return out
